# baseline (speedup 1.0000x reference)
import functools
import math

import jax
import jax.numpy as jnp
from jax.experimental import pallas as pl
from jax.experimental.pallas import tpu as pltpu

_NUM_POS_FEATS = 384
_TEMPERATURE = 10000.0
_BLK = 256


def _pos_embed_block(o_ref, sinb_ref, cosb_ref, cache_ref):
    i = pl.program_id(0)
    b = pl.program_id(1)
    blk = o_ref.shape[1]
    nf = _NUM_POS_FEATS
    kblk = blk // 2

    @pl.when((i == 0) & (b == 0))
    def _init():
        j = jax.lax.broadcasted_iota(jnp.int32, (blk, nf), 1)
        inv_dim_t = jnp.exp(
            (-math.log(_TEMPERATURE) * 2.0 / nf) * ((j // 2).astype(jnp.float32))
        )
        p = jax.lax.broadcasted_iota(jnp.int32, (blk, nf), 0)
        base = (p // 2).astype(jnp.float32) * inv_dim_t
        sinb_ref[...] = jnp.sin(base)
        cosb_ref[...] = jnp.cos(base)

    @pl.when(b == 0)
    def _compute():
        j = jax.lax.broadcasted_iota(jnp.int32, (blk, nf), 1)
        inv_dim_t = jnp.exp(
            (-math.log(_TEMPERATURE) * 2.0 / nf) * ((j // 2).astype(jnp.float32))
        )
        phase_a = (i * kblk) * inv_dim_t[:8, :]
        sin_a = jnp.sin(phase_a)[:1]
        cos_a = jnp.cos(phase_a)[:1]
        sinb = sinb_ref[...]
        cosb = cosb_ref[...]
        sin_k = sinb * cos_a + cosb * sin_a
        cos_k = cosb * cos_a - sinb * sin_a
        p = jax.lax.broadcasted_iota(jnp.int32, (blk, nf), 0)
        even = (p % 2) == 0
        sin_half = jnp.where(even, sin_k, 0.0)
        cos_half = jnp.where(even, 0.0, cos_k)
        cache_ref[...] = jnp.concatenate([sin_half, cos_half], axis=1)

    o_ref[...] = cache_ref[...][None]


@functools.partial(jax.jit, static_argnames=())
def kernel(x):
    batch, seq = x.shape
    nf2 = 2 * _NUM_POS_FEATS
    grid = (seq // _BLK, batch)
    return pl.pallas_call(
        _pos_embed_block,
        grid=grid,
        out_shape=jax.ShapeDtypeStruct((batch, seq, nf2), jnp.float32),
        out_specs=pl.BlockSpec((1, _BLK, nf2), lambda i, b: (b, i, 0)),
        scratch_shapes=[
            pltpu.VMEM((_BLK, _NUM_POS_FEATS), jnp.float32),
            pltpu.VMEM((_BLK, _NUM_POS_FEATS), jnp.float32),
            pltpu.VMEM((_BLK, 2 * _NUM_POS_FEATS), jnp.float32),
        ],
    )()


# speedup vs baseline: 2.0705x; 2.0705x over previous
import functools
import math

import jax
import jax.numpy as jnp
from jax.experimental import pallas as pl
from jax.experimental.pallas import tpu as pltpu

_NUM_POS_FEATS = 384
_TEMPERATURE = 10000.0
_BLK = 256


def _pos_embed_block(o_ref, sinb_ref, cosb_ref):
    i = pl.program_id(0)
    batch, blk, _ = o_ref.shape
    nf = _NUM_POS_FEATS
    kblk = blk // 2

    j = jax.lax.broadcasted_iota(jnp.int32, (blk, nf), 1)
    inv_dim_t = jnp.exp(
        (-math.log(_TEMPERATURE) * 2.0 / nf) * ((j // 2).astype(jnp.float32))
    )

    @pl.when(i == 0)
    def _init():
        p = jax.lax.broadcasted_iota(jnp.int32, (blk, nf), 0)
        base = (p // 2).astype(jnp.float32) * inv_dim_t
        sinb_ref[...] = jnp.sin(base)
        cosb_ref[...] = jnp.cos(base)

    phase_a = (i * kblk) * inv_dim_t[:8, :]
    sin_a = jnp.sin(phase_a)[:1]
    cos_a = jnp.cos(phase_a)[:1]

    sinb = sinb_ref[...]
    cosb = cosb_ref[...]
    sin_k = sinb * cos_a + cosb * sin_a
    cos_k = cosb * cos_a - sinb * sin_a

    p = jax.lax.broadcasted_iota(jnp.int32, (blk, nf), 0)
    even = (p % 2) == 0
    sin_half = jnp.where(even, sin_k, 0.0)
    cos_half = jnp.where(even, 0.0, cos_k)
    full = jnp.concatenate([sin_half, cos_half], axis=1)
    o_ref[...] = jnp.broadcast_to(full[None], (batch, blk, 2 * nf))


@functools.partial(jax.jit, static_argnames=())
def kernel(x):
    batch, seq = x.shape
    nf2 = 2 * _NUM_POS_FEATS
    grid = (seq // _BLK,)
    return pl.pallas_call(
        _pos_embed_block,
        grid=grid,
        out_shape=jax.ShapeDtypeStruct((batch, seq, nf2), jnp.float32),
        out_specs=pl.BlockSpec((batch, _BLK, nf2), lambda i: (0, i, 0)),
        scratch_shapes=[
            pltpu.VMEM((_BLK, _NUM_POS_FEATS), jnp.float32),
            pltpu.VMEM((_BLK, _NUM_POS_FEATS), jnp.float32),
        ],
    )()
